# baseline (speedup 1.0000x reference)
import functools

import jax
import jax.numpy as jnp
from jax import lax
from jax.experimental import pallas as pl
from jax.experimental.pallas import tpu as pltpu
from jax.experimental.pallas import tpu_sc as plsc

N = 10000
E = 320000
D = 128
H = 128
C = 7
CPAD = 16

NC = 2
NS = 16
NW = NC * NS
EPT = E // NW
CHUNK = 100
NCHUNK = EPT // CHUNK
NGRP = 10
GCH = NCHUNK // NGRP
ROWS_PT = N // NS

BM = 1000


def _mm1_body(x_ref, w_ref, o_ref):
    o_ref[...] = jnp.dot(x_ref[...], w_ref[...],
                         preferred_element_type=jnp.float32)


def _tc_mm1(x, w):
    ho = w.shape[1]
    return pl.pallas_call(
        _mm1_body,
        grid=(N // BM,),
        in_specs=[pl.BlockSpec((BM, D), lambda i: (i, 0)),
                  pl.BlockSpec((D, ho), lambda i: (0, 0))],
        out_specs=pl.BlockSpec((BM, ho), lambda i: (i, 0)),
        out_shape=jax.ShapeDtypeStruct((N, ho), jnp.float32),
    )(x, w)


def _mm_epi_body(acc_ref, b_ref, w_ref, o_ref):
    h = jnp.maximum(acc_ref[0] + acc_ref[1] + b_ref[...], 0.0)
    o_ref[...] = jnp.dot(h, w_ref[...], preferred_element_type=jnp.float32)


def _tc_mm_epi(acc, b_row, w):
    hi = acc.shape[2]
    ho = w.shape[1]
    return pl.pallas_call(
        _mm_epi_body,
        grid=(N // BM,),
        in_specs=[pl.BlockSpec((2, BM, hi), lambda i: (0, i, 0)),
                  pl.BlockSpec((1, hi), lambda i: (0, 0)),
                  pl.BlockSpec((hi, ho), lambda i: (0, 0))],
        out_specs=pl.BlockSpec((BM, ho), lambda i: (i, 0)),
        out_shape=jax.ShapeDtypeStruct((N, ho), jnp.float32),
    )(acc, b_row, w)


def _relu_body(acc_ref, b_ref, o_ref):
    o_ref[...] = jnp.maximum(acc_ref[0] + acc_ref[1] + b_ref[...], 0.0)


def _tc_relu(acc, b_row):
    return pl.pallas_call(
        _relu_body,
        grid=(N // BM,),
        in_specs=[pl.BlockSpec((2, BM, H), lambda i: (0, i, 0)),
                  pl.BlockSpec((1, H), lambda i: (0, 0))],
        out_specs=pl.BlockSpec((BM, H), lambda i: (i, 0)),
        out_shape=jax.ShapeDtypeStruct((N, H), jnp.float32),
    )(acc, b_row)


def _final_body(acc_ref, w_ref, b_ref, o_ref):
    agg = acc_ref[0] + acc_ref[1]
    t = jnp.dot(agg, w_ref[...],
                preferred_element_type=jnp.float32) + b_ref[...]
    col = lax.broadcasted_iota(jnp.int32, t.shape, 1)
    valid = col < C
    tm = jnp.where(valid, t, jnp.float32(-1e30))
    m = jnp.max(tm, axis=1, keepdims=True)
    e = jnp.where(valid, jnp.exp(tm - m), 0.0)
    s = jnp.sum(e, axis=1, keepdims=True)
    o_ref[...] = t - m - jnp.log(s)


def _tc_final(acc, w, b_row):
    return pl.pallas_call(
        _final_body,
        grid=(N // BM,),
        in_specs=[pl.BlockSpec((2, BM, H), lambda i: (0, i, 0)),
                  pl.BlockSpec((H, CPAD), lambda i: (0, 0)),
                  pl.BlockSpec((1, CPAD), lambda i: (0, 0))],
        out_specs=pl.BlockSpec((BM, CPAD), lambda i: (i, 0)),
        out_shape=jax.ShapeDtypeStruct((N, CPAD), jnp.float32),
    )(acc, w, b_row)


def _spmm(sup, src2d, dst2d, vals2d, zeros):
    hs = sup.shape[1]
    nvec = hs // 16
    mesh = plsc.VectorSubcoreMesh(core_axis_name="c", subcore_axis_name="s")

    @functools.partial(
        pl.kernel,
        out_type=jax.ShapeDtypeStruct((NC, NS, ROWS_PT, hs), jnp.float32),
        mesh=mesh,
        scratch_types=[
            pltpu.VMEM((GCH, CHUNK), jnp.int32),
            pltpu.VMEM((GCH, CHUNK), jnp.int32),
            pltpu.VMEM((GCH, CHUNK), jnp.float32),
            pltpu.VMEM((CHUNK, hs), jnp.float32),
            pltpu.VMEM((CHUNK, hs), jnp.float32),
            pltpu.VMEM_SHARED((N, hs), jnp.float32),
            pltpu.SemaphoreType.DMA,
            pltpu.SemaphoreType.DMA,
            pltpu.SemaphoreType.DMA,
            pltpu.SemaphoreType.DMA,
        ],
    )
    def k(sup_hbm, src_hbm, dst_hbm, vals_hbm, zeros_hbm, out_hbm,
          src_v, dst_v, vals_v, rows0_v, rows1_v, acc_sh,
          sem0, sem1, sem2, sem3):
        c = lax.axis_index("c")
        s = lax.axis_index("s")
        wid = c * NS + s

        pltpu.sync_copy(zeros_hbm.at[s],
                        acc_sh.at[pl.ds(s * ROWS_PT, ROWS_PT)])
        plsc.subcore_barrier()

        def start_gather(jj, buf, sem):
            pltpu.async_copy(sup_hbm.at[src_v.at[jj]], buf, sem)

        def wait_gather(buf, sem):
            pltpu.make_async_copy(sup_hbm.at[src_v.at[0]], buf, sem).wait()

        def start_scatter(jj, buf, sem):
            pltpu.async_copy(buf, acc_sh.at[dst_v.at[jj]], sem, add=True)

        def wait_scatter(buf, sem):
            pltpu.make_async_copy(buf, acc_sh.at[dst_v.at[0]], sem).wait()

        def scale(buf, jj):
            ngroup = CHUNK // 16
            ntail = CHUNK - ngroup * 16
            starts = [q * 16 for q in range(ngroup)]
            lanes = [list(range(16))] * ngroup
            if ntail:
                starts.append(CHUNK - 16)
                lanes.append(list(range(16 - ntail, 16)))
            edge = 0
            for start, lns in zip(starts, lanes):
                vv = vals_v[jj, pl.ds(start, 16)]
                bcs = [jnp.full((16,), vv[e], jnp.float32) for e in lns]
                for b, bc in enumerate(bcs):
                    i = edge + b
                    for f in range(nvec):
                        sl = pl.ds(f * 16, 16)
                        buf[i, sl] = buf[i, sl] * bc
                edge += len(lns)

        def group_body(g, carry_g):
            pltpu.sync_copy(src_hbm.at[wid, g], src_v)
            pltpu.sync_copy(dst_hbm.at[wid, g], dst_v)
            pltpu.sync_copy(vals_hbm.at[wid, g], vals_v)
            start_gather(0, rows0_v, sem0)

            def pair_body(m, carry):
                j0 = 2 * m

                @pl.when(m > 0)
                def _():
                    wait_scatter(rows1_v, sem3)

                start_gather(j0 + 1, rows1_v, sem1)
                wait_gather(rows0_v, sem0)
                scale(rows0_v, j0)
                start_scatter(j0, rows0_v, sem2)
                wait_gather(rows1_v, sem1)
                scale(rows1_v, j0 + 1)
                wait_scatter(rows0_v, sem2)

                @pl.when(m + 1 < GCH // 2)
                def _():
                    start_gather(j0 + 2, rows0_v, sem0)

                start_scatter(j0 + 1, rows1_v, sem3)
                return carry

            lax.fori_loop(0, GCH // 2, pair_body, 0)
            wait_scatter(rows1_v, sem3)
            return carry_g

        lax.fori_loop(0, NGRP, group_body, 0)

        plsc.subcore_barrier()
        pltpu.sync_copy(acc_sh.at[pl.ds(s * ROWS_PT, ROWS_PT)],
                        out_hbm.at[c, s])

    return k(sup, src2d, dst2d, vals2d, zeros).reshape(NC, N, hs)


def kernel(x, adj_indices, adj_values, W1, b1, W2, b2, W3, b3, W4, b4):
    src2d = adj_indices[0].reshape(NW, NGRP, GCH, CHUNK)
    dst2d = adj_indices[1].reshape(NW, NGRP, GCH, CHUNK)
    vals2d = adj_values.reshape(NW, NGRP, GCH, CHUNK)
    zeros_h = jnp.zeros((NS, ROWS_PT, H), jnp.float32)
    w4p = jnp.pad(W4, ((0, 0), (0, CPAD - C)))
    b4p = jnp.pad(b4, (0, CPAD - C)).reshape(1, CPAD)

    sup1 = _tc_mm1(x, W1)
    acc1 = _spmm(sup1, src2d, dst2d, vals2d, zeros_h)
    sup2 = _tc_mm_epi(acc1, b1.reshape(1, H), W2)
    acc2 = _spmm(sup2, src2d, dst2d, vals2d, zeros_h)
    sup3 = _tc_mm_epi(acc2, b2.reshape(1, H), W3)
    acc3 = _spmm(sup3, src2d, dst2d, vals2d, zeros_h)
    h3 = _tc_relu(acc3, b3.reshape(1, H))
    acc4 = _spmm(h3, src2d, dst2d, vals2d, zeros_h)
    outp = _tc_final(acc4, w4p, b4p)
    return outp[:, :C]


# speedup vs baseline: 1.0360x; 1.0360x over previous
import functools

import jax
import jax.numpy as jnp
from jax import lax
from jax.experimental import pallas as pl
from jax.experimental.pallas import tpu as pltpu
from jax.experimental.pallas import tpu_sc as plsc

N = 10000
E = 320000
D = 128
H = 128
C = 7
CPAD = 16

NC = 2
NS = 16
NW = NC * NS
EPT = E // NW
CHUNK = 100
NCHUNK = EPT // CHUNK
NGRP = 10
GCH = NCHUNK // NGRP
ROWS_PT = N // NS

BM = 1000


def _mm1_body(x_ref, w_ref, o_ref):
    o_ref[...] = jnp.dot(x_ref[...], w_ref[...],
                         preferred_element_type=jnp.float32)


def _tc_mm1(x, w):
    ho = w.shape[1]
    return pl.pallas_call(
        _mm1_body,
        grid=(N // BM,),
        in_specs=[pl.BlockSpec((BM, D), lambda i: (i, 0)),
                  pl.BlockSpec((D, ho), lambda i: (0, 0))],
        out_specs=pl.BlockSpec((BM, ho), lambda i: (i, 0)),
        out_shape=jax.ShapeDtypeStruct((N, ho), jnp.float32),
    )(x, w)


def _mm_epi_body(acc_ref, b_ref, w_ref, o_ref):
    h = jnp.maximum(acc_ref[0] + acc_ref[1] + b_ref[...], 0.0)
    o_ref[...] = jnp.dot(h, w_ref[...], preferred_element_type=jnp.float32)


def _tc_mm_epi(acc, b_row, w):
    hi = acc.shape[2]
    ho = w.shape[1]
    return pl.pallas_call(
        _mm_epi_body,
        grid=(N // BM,),
        in_specs=[pl.BlockSpec((2, BM, hi), lambda i: (0, i, 0)),
                  pl.BlockSpec((1, hi), lambda i: (0, 0)),
                  pl.BlockSpec((hi, ho), lambda i: (0, 0))],
        out_specs=pl.BlockSpec((BM, ho), lambda i: (i, 0)),
        out_shape=jax.ShapeDtypeStruct((N, ho), jnp.float32),
    )(acc, b_row, w)


def _relu_body(acc_ref, b_ref, o_ref):
    o_ref[...] = jnp.maximum(acc_ref[0] + acc_ref[1] + b_ref[...], 0.0)


def _tc_relu(acc, b_row):
    return pl.pallas_call(
        _relu_body,
        grid=(N // BM,),
        in_specs=[pl.BlockSpec((2, BM, H), lambda i: (0, i, 0)),
                  pl.BlockSpec((1, H), lambda i: (0, 0))],
        out_specs=pl.BlockSpec((BM, H), lambda i: (i, 0)),
        out_shape=jax.ShapeDtypeStruct((N, H), jnp.float32),
    )(acc, b_row)


def _final_body(acc_ref, w_ref, b_ref, o_ref):
    agg = acc_ref[0] + acc_ref[1]
    t = jnp.dot(agg, w_ref[...],
                preferred_element_type=jnp.float32) + b_ref[...]
    col = lax.broadcasted_iota(jnp.int32, t.shape, 1)
    valid = col < C
    tm = jnp.where(valid, t, jnp.float32(-1e30))
    m = jnp.max(tm, axis=1, keepdims=True)
    e = jnp.where(valid, jnp.exp(tm - m), 0.0)
    s = jnp.sum(e, axis=1, keepdims=True)
    o_ref[...] = t - m - jnp.log(s)


def _tc_final(acc, w, b_row):
    return pl.pallas_call(
        _final_body,
        grid=(N // BM,),
        in_specs=[pl.BlockSpec((2, BM, H), lambda i: (0, i, 0)),
                  pl.BlockSpec((H, CPAD), lambda i: (0, 0)),
                  pl.BlockSpec((1, CPAD), lambda i: (0, 0))],
        out_specs=pl.BlockSpec((BM, CPAD), lambda i: (i, 0)),
        out_shape=jax.ShapeDtypeStruct((N, CPAD), jnp.float32),
    )(acc, w, b_row)


def _spmm(sup, src2d, dst2d, vals2d, zeros):
    hs = sup.shape[1]
    nvec = hs // 16
    mesh = plsc.VectorSubcoreMesh(core_axis_name="c", subcore_axis_name="s")

    @functools.partial(
        pl.kernel,
        out_type=jax.ShapeDtypeStruct((NC, NS, ROWS_PT, hs), jnp.float32),
        mesh=mesh,
        scratch_types=[
            pltpu.VMEM((GCH, CHUNK), jnp.int32),
            pltpu.VMEM((GCH, CHUNK), jnp.int32),
            pltpu.VMEM((GCH, CHUNK), jnp.float32),
            pltpu.VMEM((CHUNK, hs), jnp.float32),
            pltpu.VMEM((CHUNK, hs), jnp.float32),
            pltpu.VMEM_SHARED((N, hs), jnp.float32),
            pltpu.SemaphoreType.DMA,
            pltpu.SemaphoreType.DMA,
            pltpu.SemaphoreType.DMA,
            pltpu.SemaphoreType.DMA,
        ],
    )
    def k(sup_hbm, src_hbm, dst_hbm, vals_hbm, zeros_hbm, out_hbm,
          src_v, dst_v, vals_v, rows0_v, rows1_v, acc_sh,
          sem0, sem1, sem2, sem3):
        c = lax.axis_index("c")
        s = lax.axis_index("s")
        wid = c * NS + s

        pltpu.sync_copy(zeros_hbm.at[s],
                        acc_sh.at[pl.ds(s * ROWS_PT, ROWS_PT)])
        plsc.subcore_barrier()

        def start_gather(jj, buf, sem):
            pltpu.async_copy(sup_hbm.at[src_v.at[jj]], buf, sem)

        def wait_gather(buf, sem):
            pltpu.make_async_copy(sup_hbm.at[src_v.at[0]], buf, sem).wait()

        def start_scatter(jj, buf, sem):
            pltpu.async_copy(buf, acc_sh.at[dst_v.at[jj]], sem, add=True)

        def wait_scatter(buf, sem):
            pltpu.make_async_copy(buf, acc_sh.at[dst_v.at[0]], sem).wait()

        def scale(buf, jj):
            ngroup = CHUNK // 16
            ntail = CHUNK - ngroup * 16
            starts = [q * 16 for q in range(ngroup)]
            lanes = [list(range(16))] * ngroup
            if ntail:
                starts.append(CHUNK - 16)
                lanes.append(list(range(16 - ntail, 16)))
            edge = 0
            for start, lns in zip(starts, lanes):
                vv = vals_v[jj, pl.ds(start, 16)]
                bcs = [jnp.full((16,), vv[e], jnp.float32) for e in lns]
                for b, bc in enumerate(bcs):
                    i = edge + b
                    for f in range(nvec):
                        sl = pl.ds(f * 16, 16)
                        buf[i, sl] = buf[i, sl] * bc
                edge += len(lns)

        def group_body(g, carry_g):
            pltpu.sync_copy(src_hbm.at[wid, g], src_v)
            pltpu.sync_copy(dst_hbm.at[wid, g], dst_v)
            pltpu.sync_copy(vals_hbm.at[wid, g], vals_v)
            start_gather(0, rows0_v, sem0)

            def pair_body(m, carry):
                j0 = 2 * m
                start_gather(j0 + 1, rows1_v, sem1)
                wait_gather(rows0_v, sem0)
                scale(rows0_v, j0)
                pltpu.sync_copy(rows0_v, acc_sh.at[dst_v.at[j0]], add=True)

                @pl.when(m + 1 < GCH // 2)
                def _():
                    start_gather(j0 + 2, rows0_v, sem0)

                wait_gather(rows1_v, sem1)
                scale(rows1_v, j0 + 1)
                pltpu.sync_copy(rows1_v, acc_sh.at[dst_v.at[j0 + 1]], add=True)
                return carry

            lax.fori_loop(0, GCH // 2, pair_body, 0)
            return carry_g

        lax.fori_loop(0, NGRP, group_body, 0)

        plsc.subcore_barrier()
        pltpu.sync_copy(acc_sh.at[pl.ds(s * ROWS_PT, ROWS_PT)],
                        out_hbm.at[c, s])

    return k(sup, src2d, dst2d, vals2d, zeros).reshape(NC, N, hs)


def kernel(x, adj_indices, adj_values, W1, b1, W2, b2, W3, b3, W4, b4):
    src2d = adj_indices[0].reshape(NW, NGRP, GCH, CHUNK)
    dst2d = adj_indices[1].reshape(NW, NGRP, GCH, CHUNK)
    vals2d = adj_values.reshape(NW, NGRP, GCH, CHUNK)
    zeros_h = jnp.zeros((NS, ROWS_PT, H), jnp.float32)
    w4p = jnp.pad(W4, ((0, 0), (0, CPAD - C)))
    b4p = jnp.pad(b4, (0, CPAD - C)).reshape(1, CPAD)

    sup1 = _tc_mm1(x, W1)
    acc1 = _spmm(sup1, src2d, dst2d, vals2d, zeros_h)
    sup2 = _tc_mm_epi(acc1, b1.reshape(1, H), W2)
    acc2 = _spmm(sup2, src2d, dst2d, vals2d, zeros_h)
    sup3 = _tc_mm_epi(acc2, b2.reshape(1, H), W3)
    acc3 = _spmm(sup3, src2d, dst2d, vals2d, zeros_h)
    h3 = _tc_relu(acc3, b3.reshape(1, H))
    acc4 = _spmm(h3, src2d, dst2d, vals2d, zeros_h)
    outp = _tc_final(acc4, w4p, b4p)
    return outp[:, :C]
